# baseline (speedup 1.0000x reference)
import jax
import jax.numpy as jnp
from jax import lax
from jax.experimental import pallas as pl
from jax.experimental.pallas import tpu as pltpu
from jax.experimental.pallas import tpu_sc as plsc

VOCAB_E = 100000
EMBED_E = 128
B_E = 4096
L_E = 50

NC = 2
NS = 16
NW = NC * NS

PER_B = B_E // NW
CB = 8
CH = CB * L_E
NCHUNK = PER_B // CB
NSLOT = 2


def _emb_body(tokens_hbm, table_hbm, out_hbm, idx0, idx1, rows0, rows1,
              isem0, isem1, gsem0, gsem1, osem0, osem1):
  wid = lax.axis_index("s") * NC + lax.axis_index("c")
  bbase = wid * PER_B

  idx = (idx0, idx1)
  rows = (rows0, rows1)
  isem = (isem0, isem1)
  gsem = (gsem0, gsem1)
  osem = (osem0, osem1)

  def drain_writes(b):
    for _ in range(CB):
      pltpu.make_async_copy(rows[b].at[pl.ds(0, L_E)], out_hbm.at[0],
                            osem[b]).wait()

  def write_back(c):
    b = c % NSLOT
    pltpu.make_async_copy(table_hbm.at[idx[b]], rows[b], gsem[b]).wait()
    for i in range(CB):
      pltpu.async_copy(rows[b].at[pl.ds(i * L_E, L_E)],
                       out_hbm.at[bbase + c * CB + i], osem[b])
    if c + NSLOT < NCHUNK:
      pltpu.async_copy(tokens_hbm.at[wid, c + NSLOT], idx[b], isem[b])

  for b in range(NSLOT):
    pltpu.async_copy(tokens_hbm.at[wid, b], idx[b], isem[b])

  for c in range(NCHUNK):
    b = c % NSLOT
    pltpu.make_async_copy(tokens_hbm.at[wid, c], idx[b], isem[b]).wait()
    if c >= NSLOT:
      drain_writes(b)
    pltpu.async_copy(table_hbm.at[idx[b]], rows[b], gsem[b])
    if c >= 1:
      write_back(c - 1)

  write_back(NCHUNK - 1)
  for b in range(NSLOT):
    drain_writes(b)


@jax.jit
def _embed(tokens_flat, table):
  k = pl.kernel(
      _emb_body,
      out_type=jax.ShapeDtypeStruct((B_E, L_E, EMBED_E), jnp.float32),
      mesh=plsc.VectorSubcoreMesh(core_axis_name="c", subcore_axis_name="s"),
      scratch_types=(
          [pltpu.VMEM((CH,), jnp.int32) for _ in range(NSLOT)]
          + [pltpu.VMEM((CH, EMBED_E), jnp.float32) for _ in range(NSLOT)]
          + [pltpu.SemaphoreType.DMA] * (3 * NSLOT)
      ),
      compiler_params=pltpu.CompilerParams(use_tc_tiling_on_sc=True),
  )
  return k(tokens_flat, table)


def kernel(tokens, table):
  tokens_flat = tokens.astype(jnp.int32).reshape(NW, NCHUNK, CH)
  return _embed(tokens_flat, table)


# speedup vs baseline: 1.7759x; 1.7759x over previous
import jax
import jax.numpy as jnp
from jax import lax
from jax.experimental import pallas as pl
from jax.experimental.pallas import tpu as pltpu
from jax.experimental.pallas import tpu_sc as plsc

VOCAB_E = 100000
EMBED_E = 128
B_E = 4096
L_E = 50

NC = 2
NS = 16
NW = NC * NS

N_TOK = B_E * L_E
PER_W = N_TOK // NW
CH = 400
NCHUNK = PER_W // CH
NSLOT = 2


def _emb_body(tokens_hbm, table_hbm, out_hbm, idx0, idx1, rows0, rows1,
              isem0, isem1, gsem0, gsem1, osem0, osem1):
  wid = lax.axis_index("s") * NC + lax.axis_index("c")
  base = wid * PER_W

  idx = (idx0, idx1)
  rows = (rows0, rows1)
  isem = (isem0, isem1)
  gsem = (gsem0, gsem1)
  osem = (osem0, osem1)

  def write_back(c):
    b = c % NSLOT
    pltpu.make_async_copy(table_hbm.at[idx[b]], rows[b], gsem[b]).wait()
    pltpu.async_copy(rows[b], out_hbm.at[pl.ds(base + c * CH, CH)], osem[b])
    if c + NSLOT < NCHUNK:
      pltpu.async_copy(tokens_hbm.at[wid, c + NSLOT], idx[b], isem[b])

  for b in range(NSLOT):
    pltpu.async_copy(tokens_hbm.at[wid, b], idx[b], isem[b])

  for c in range(NCHUNK):
    b = c % NSLOT
    pltpu.make_async_copy(tokens_hbm.at[wid, c], idx[b], isem[b]).wait()
    if c >= NSLOT:
      pltpu.make_async_copy(rows[b], out_hbm.at[pl.ds(0, CH)], osem[b]).wait()
    pltpu.async_copy(table_hbm.at[idx[b]], rows[b], gsem[b])
    if c >= 1:
      write_back(c - 1)

  write_back(NCHUNK - 1)
  for b in range(NSLOT):
    pltpu.make_async_copy(rows[b], out_hbm.at[pl.ds(0, CH)], osem[b]).wait()


@jax.jit
def _embed(tokens_flat, table):
  k = pl.kernel(
      _emb_body,
      out_type=jax.ShapeDtypeStruct((N_TOK, EMBED_E), jnp.float32),
      mesh=plsc.VectorSubcoreMesh(core_axis_name="c", subcore_axis_name="s"),
      scratch_types=(
          [pltpu.VMEM((CH,), jnp.int32) for _ in range(NSLOT)]
          + [pltpu.VMEM((CH, EMBED_E), jnp.float32) for _ in range(NSLOT)]
          + [pltpu.SemaphoreType.DMA] * (3 * NSLOT)
      ),
  )
  return k(tokens_flat, table)


def kernel(tokens, table):
  tokens_t = jnp.transpose(tokens).astype(jnp.int32).reshape(NW, NCHUNK, CH)
  out = _embed(tokens_t, table)
  return jnp.transpose(out.reshape(L_E, B_E, EMBED_E), (1, 0, 2))
